# Initial kernel scaffold; baseline (speedup 1.0000x reference)
import jax
import jax.numpy as jnp
from jax.experimental import pallas as pl


def kernel(inputs, wt_pers):
    raise NotImplementedError("write your pallas kernel here")


# speedup vs baseline: 1.8393x; 1.8393x over previous
import jax
import jax.numpy as jnp
from jax import lax
from jax.experimental import pallas as pl
from jax.experimental.pallas import tpu as pltpu
from jax.experimental.pallas import tpu_sc as plsc

_NC = 2
_NS = 16
_NW = _NC * _NS
_L = 16


def _make_body(TM, C, per, H, W, RC):
    NQ = W // _L
    NCH = H // RC
    fx1 = float(W - 2)
    fy1 = float(H - 2)

    def body(imgs, ths, grid, out, img_v, th_v, g_v, ob_v):
        wid = lax.axis_index("s") * _NC + lax.axis_index("c")
        pltpu.sync_copy(grid, g_v)
        pltpu.sync_copy(ths, th_v)

        def do_image(k, carry):
            q = wid * per + k
            img_idx = q // TM
            tt = q - img_idx * TM
            c = img_idx % C
            th_idx = tt * C + c
            pltpu.sync_copy(imgs.at[img_idx], img_v)
            thi = jnp.full((_L,), th_idx, dtype=jnp.int32)
            tv = [plsc.load_gather(th_v, [thi, jnp.full((_L,), kk, dtype=jnp.int32)])
                  for kk in range(8)]
            onev = jnp.full((_L,), 1.0, dtype=jnp.float32)

            def do_chunk(cc, carry2):
                def do_row(r, carry3):
                    i = cc * RC + r
                    xgv = plsc.load_gather(g_v, [jnp.full((_L,), i, dtype=jnp.int32)])
                    av = tv[0] * xgv + tv[2]
                    bv = tv[3] * xgv + tv[5]
                    cv = tv[6] * xgv + onev
                    for qq in range(NQ):
                        j0 = qq * _L
                        yv = g_v[pl.ds(j0, _L)]
                        xl = av + tv[1] * yv
                        yl = bv + tv[4] * yv
                        om = cv + tv[7] * yv
                        xs = xl / om
                        ys = yl / om
                        xm = ((xs + 1.0) * fx1) * 0.5
                        ym = ((ys + 1.0) * fy1) * 0.5
                        x0 = xm.astype(jnp.int32)
                        x0 = jnp.where(x0.astype(jnp.float32) > xm, x0 - 1, x0)
                        y0 = ym.astype(jnp.int32)
                        y0 = jnp.where(y0.astype(jnp.float32) > ym, y0 - 1, y0)
                        x0c = jnp.clip(x0, 0, W - 1)
                        x1c = jnp.clip(x0 + 1, 0, W - 1)
                        y0c = jnp.clip(y0, 0, H - 1)
                        y1c = jnp.clip(y0 + 1, 0, H - 1)
                        Ia = plsc.load_gather(img_v, [y0c, x0c])
                        Ib = plsc.load_gather(img_v, [y1c, x0c])
                        Ic = plsc.load_gather(img_v, [y0c, x1c])
                        Id = plsc.load_gather(img_v, [y1c, x1c])
                        x0f = x0c.astype(jnp.float32)
                        x1f = x1c.astype(jnp.float32)
                        y0f = y0c.astype(jnp.float32)
                        y1f = y1c.astype(jnp.float32)
                        wa = (x1f - xm) * (y1f - ym)
                        wb = (x1f - xm) * (y1f - y0f)
                        wc = (xm - x0f) * (y1f - ym)
                        wd = (xm - x0f) * (ym - y0f)
                        val = wa * Ia + wb * Ib + wc * Ic + wd * Id
                        ob_v[r, pl.ds(j0, _L)] = val
                    return carry3
                lax.fori_loop(0, RC, do_row, carry2)
                pltpu.sync_copy(ob_v, out.at[q, pl.ds(cc * RC, RC)])
                return carry2
            lax.fori_loop(0, NCH, do_chunk, carry)
            return carry
        lax.fori_loop(0, per, do_image, 0)

    return body


def kernel(inputs, wt_pers):
    B, C, H, W = inputs.shape
    TM = wt_pers.shape[0]
    ntasks = B * C * TM
    per = ntasks // _NW
    RC = 8
    imgs = inputs.reshape(B * C, H, W)
    ths = wt_pers.reshape(TM * C, 8)
    ths = ths.astype(jnp.bfloat16).astype(jnp.float32)
    grid = jnp.linspace(-1.0, 1.0, W, dtype=jnp.float32)
    grid = grid.astype(jnp.bfloat16).astype(jnp.float32)
    kfn = pl.kernel(
        _make_body(TM, C, per, H, W, RC),
        out_type=jax.ShapeDtypeStruct((ntasks, H, W), jnp.float32),
        mesh=plsc.VectorSubcoreMesh(core_axis_name="c", subcore_axis_name="s"),
        compiler_params=pltpu.CompilerParams(needs_layout_passes=False),
        scratch_types=[
            pltpu.VMEM((H, W), jnp.float32),
            pltpu.VMEM((TM * C, 8), jnp.float32),
            pltpu.VMEM((W,), jnp.float32),
            pltpu.VMEM((RC, W), jnp.float32),
        ],
    )
    out = kfn(imgs, ths, grid)
    return out.reshape(B, C * TM, H, W)


# speedup vs baseline: 2.6586x; 1.4454x over previous
import jax
import jax.numpy as jnp
from jax import lax
from jax.experimental import pallas as pl
from jax.experimental.pallas import tpu as pltpu
from jax.experimental.pallas import tpu_sc as plsc

_NC = 2
_NS = 16
_NW = _NC * _NS
_L = 16


def _floor_fix(v):
    i = v.astype(jnp.int32)
    return jnp.where(i.astype(jnp.float32) > v, i - 1, i)


def _make_body(TM, C, per, H, W, RC):
    NQ = W // _L
    NCH = H // RC
    fx1 = float(W - 2)
    fy1 = float(H - 2)

    def body(imgs, ths, grid, out,
             img_v, th_v, g_v, ob_v, y0_t, wd_t, dyc_t):
        wid = lax.axis_index("s") * _NC + lax.axis_index("c")
        pltpu.sync_copy(grid, g_v)
        pltpu.sync_copy(ths, th_v)

        def do_image(k, carry):
            q = wid * per + k
            img_idx = q // TM
            tt = q - img_idx * TM
            c = img_idx % C
            th_idx = tt * C + c
            pltpu.sync_copy(imgs.at[img_idx], img_v)
            thi = jnp.full((_L,), th_idx, dtype=jnp.int32)
            tv = [plsc.load_gather(
                      th_v, [thi, jnp.full((_L,), kk, dtype=jnp.int32)])
                  for kk in range(8)]
            onev = jnp.full((_L,), 1.0, dtype=jnp.float32)
            separable = ((tv[1][0] == 0.0) & (tv[3][0] == 0.0)
                         & (tv[6][0] == 0.0) & (tv[7][0] == 0.0))

            def row_x(i):
                xgv = plsc.load_gather(
                    g_v, [jnp.full((_L,), i, dtype=jnp.int32)])
                xsv = tv[0] * xgv + tv[2]
                xmv = ((xsv + 1.0) * fx1) * 0.5
                x0 = _floor_fix(xmv)
                x0c = jnp.clip(x0, 0, W - 1)
                x1c = jnp.clip(x0 + 1, 0, W - 1)
                x0f = x0c.astype(jnp.float32)
                x1f = x1c.astype(jnp.float32)
                return x0c, x1c, x1f - xmv, xmv - x0f

            def fast_path():
                for qq in range(NQ):
                    j0 = qq * _L
                    yv = g_v[pl.ds(j0, _L)]
                    ysv = tv[4] * yv + tv[5]
                    ymv = ((ysv + 1.0) * fy1) * 0.5
                    y0 = _floor_fix(ymv)
                    y0c = jnp.clip(y0, 0, H - 1)
                    y1c = jnp.clip(y0 + 1, 0, H - 1)
                    y0f = y0c.astype(jnp.float32)
                    y1f = y1c.astype(jnp.float32)
                    y0_t[pl.ds(j0, _L)] = y0c
                    wd_t[pl.ds(j0, _L)] = ymv - y0f
                    dyc_t[pl.ds(j0, _L)] = y1f - y0f

                def do_chunk(cc, carry2):
                    def do_row(r, carry3):
                        i = cc * RC + r
                        x0c, x1c, dxav, dxbv = row_x(i)
                        for qq in range(NQ):
                            j0 = qq * _L
                            y0cv = y0_t[pl.ds(j0, _L)]
                            wdv = wd_t[pl.ds(j0, _L)]
                            dycv = dyc_t[pl.ds(j0, _L)]
                            y1cv = jnp.minimum(y0cv + 1, H - 1)
                            wav = dycv - wdv
                            Ia = plsc.load_gather(img_v, [y0cv, x0c])
                            Ib = plsc.load_gather(img_v, [y1cv, x0c])
                            Ic = plsc.load_gather(img_v, [y0cv, x1c])
                            Id = plsc.load_gather(img_v, [y1cv, x1c])
                            u = wav * Ia + dycv * Ib
                            v = wav * Ic + wdv * Id
                            ob_v[r, pl.ds(j0, _L)] = dxav * u + dxbv * v
                        return carry3
                    lax.fori_loop(0, RC, do_row, carry2)
                    pltpu.sync_copy(ob_v, out.at[q, pl.ds(cc * RC, RC)])
                    return carry2
                lax.fori_loop(0, NCH, do_chunk, 0)

            def slow_path():
                def do_chunk(cc, carry2):
                    def do_row(r, carry3):
                        i = cc * RC + r
                        xgv = plsc.load_gather(
                            g_v, [jnp.full((_L,), i, dtype=jnp.int32)])
                        av = tv[0] * xgv + tv[2]
                        bv = tv[3] * xgv + tv[5]
                        cv = tv[6] * xgv + onev
                        for qq in range(NQ):
                            j0 = qq * _L
                            yv = g_v[pl.ds(j0, _L)]
                            xl = av + tv[1] * yv
                            yl = bv + tv[4] * yv
                            om = cv + tv[7] * yv
                            xs = xl / om
                            ys = yl / om
                            xm = ((xs + 1.0) * fx1) * 0.5
                            ym = ((ys + 1.0) * fy1) * 0.5
                            x0 = _floor_fix(xm)
                            y0 = _floor_fix(ym)
                            x0c = jnp.clip(x0, 0, W - 1)
                            x1c = jnp.clip(x0 + 1, 0, W - 1)
                            y0c = jnp.clip(y0, 0, H - 1)
                            y1c = jnp.clip(y0 + 1, 0, H - 1)
                            Ia = plsc.load_gather(img_v, [y0c, x0c])
                            Ib = plsc.load_gather(img_v, [y1c, x0c])
                            Ic = plsc.load_gather(img_v, [y0c, x1c])
                            Id = plsc.load_gather(img_v, [y1c, x1c])
                            x0f = x0c.astype(jnp.float32)
                            x1f = x1c.astype(jnp.float32)
                            y0f = y0c.astype(jnp.float32)
                            y1f = y1c.astype(jnp.float32)
                            wa = (x1f - xm) * (y1f - ym)
                            wb = (x1f - xm) * (y1f - y0f)
                            wc = (xm - x0f) * (y1f - ym)
                            wd = (xm - x0f) * (ym - y0f)
                            val = wa * Ia + wb * Ib + wc * Ic + wd * Id
                            ob_v[r, pl.ds(j0, _L)] = val
                        return carry3
                    lax.fori_loop(0, RC, do_row, carry2)
                    pltpu.sync_copy(ob_v, out.at[q, pl.ds(cc * RC, RC)])
                    return carry2
                lax.fori_loop(0, NCH, do_chunk, 0)

            pl.when(separable)(fast_path)
            pl.when(jnp.logical_not(separable))(slow_path)
            return carry
        lax.fori_loop(0, per, do_image, 0)

    return body


def kernel(inputs, wt_pers):
    B, C, H, W = inputs.shape
    TM = wt_pers.shape[0]
    ntasks = B * C * TM
    per = ntasks // _NW
    RC = 16
    imgs = inputs.reshape(B * C, H, W)
    ths = wt_pers.reshape(TM * C, 8)
    ths = ths.astype(jnp.bfloat16).astype(jnp.float32)
    grid = jnp.linspace(-1.0, 1.0, W, dtype=jnp.float32)
    grid = grid.astype(jnp.bfloat16).astype(jnp.float32)
    kfn = pl.kernel(
        _make_body(TM, C, per, H, W, RC),
        out_type=jax.ShapeDtypeStruct((ntasks, H, W), jnp.float32),
        mesh=plsc.VectorSubcoreMesh(core_axis_name="c", subcore_axis_name="s"),
        compiler_params=pltpu.CompilerParams(needs_layout_passes=False),
        scratch_types=[
            pltpu.VMEM((H, W), jnp.float32),
            pltpu.VMEM((TM * C, 8), jnp.float32),
            pltpu.VMEM((W,), jnp.float32),
            pltpu.VMEM((RC, W), jnp.float32),
            pltpu.VMEM((W,), jnp.int32),
            pltpu.VMEM((W,), jnp.float32),
            pltpu.VMEM((W,), jnp.float32),
        ],
    )
    out = kfn(imgs, ths, grid)
    return out.reshape(B, C * TM, H, W)


# speedup vs baseline: 5.5141x; 2.0741x over previous
import jax
import jax.numpy as jnp
from jax import lax
from jax.experimental import pallas as pl
from jax.experimental.pallas import tpu as pltpu
from jax.experimental.pallas import tpu_sc as plsc

_NC = 2
_NS = 16
_NW = _NC * _NS
_L = 16


def _floor_fix(v):
    i = v.astype(jnp.int32)
    return jnp.where(i.astype(jnp.float32) > v, i - 1, i)


def _make_body(TM, C, per, H, W, RC):
    NQ = W // _L
    NCH = H // RC
    NCH2 = NCH // 2
    fx1 = float(W - 2)
    fy1 = float(H - 2)
    CB = RC * W

    def body(imgs, ths, grid, out,
             img_v, th_v, g_v, ob0, ob1,
             x0_t, x1_t, dxa_t, dxb_t, yb0_t, wd_t, dyc_t,
             sem0, sem1):
        wid = lax.axis_index("s") * _NC + lax.axis_index("c")
        pltpu.sync_copy(grid, g_v)
        pltpu.sync_copy(ths, th_v)

        def splat(val_i32):
            return jnp.full((_L,), val_i32, dtype=jnp.int32)

        def do_image(k, carry):
            q = wid * per + k
            img_idx = q // TM
            tt = q - img_idx * TM
            c = img_idx % C
            th_idx = tt * C + c
            pltpu.sync_copy(imgs.at[img_idx], img_v)
            thi = splat(th_idx)
            tv = [plsc.load_gather(th_v, [thi, splat(kk)]) for kk in range(8)]
            onev = jnp.full((_L,), 1.0, dtype=jnp.float32)
            separable = ((tv[1][0] == 0.0) & (tv[3][0] == 0.0)
                         & (tv[6][0] == 0.0) & (tv[7][0] == 0.0))

            def store_chunk(ob, sem, cc, first):
                hbm = out.at[q, pl.ds(cc * CB, CB)]
                pl.when(jnp.logical_not(first))(
                    lambda: pltpu.make_async_copy(ob, hbm, sem).wait())
                return hbm

            def fast_path():
                for qq in range(NQ):
                    j0 = qq * _L
                    gv = g_v[pl.ds(j0, _L)]
                    xsv = tv[0] * gv + tv[2]
                    xmv = ((xsv + 1.0) * fx1) * 0.5
                    x0 = _floor_fix(xmv)
                    x0c = jnp.clip(x0, 0, W - 1)
                    x1c = jnp.clip(x0 + 1, 0, W - 1)
                    x0_t[pl.ds(j0, _L)] = x0c
                    x1_t[pl.ds(j0, _L)] = x1c
                    dxa_t[pl.ds(j0, _L)] = x1c.astype(jnp.float32) - xmv
                    dxb_t[pl.ds(j0, _L)] = xmv - x0c.astype(jnp.float32)
                    ysv = tv[4] * gv + tv[5]
                    ymv = ((ysv + 1.0) * fy1) * 0.5
                    y0 = _floor_fix(ymv)
                    y0c = jnp.clip(y0, 0, H - 1)
                    y1c = jnp.clip(y0 + 1, 0, H - 1)
                    y0f = y0c.astype(jnp.float32)
                    y1f = y1c.astype(jnp.float32)
                    yb0_t[pl.ds(j0, _L)] = y0c * W
                    wd_t[pl.ds(j0, _L)] = ymv - y0f
                    dyc_t[pl.ds(j0, _L)] = y1f - y0f

                def do_pair(cc2, carry2):
                    for half, (ob, sem) in enumerate(((ob0, sem0),
                                                      (ob1, sem1))):
                        cc = cc2 * 2 + half
                        first = (k == 0) & (cc2 == 0)
                        hbm = store_chunk(ob, sem, cc, first)

                        @plsc.parallel_loop(0, RC, unroll=2)
                        def do_row(r):
                            i = cc * RC + r
                            iv = splat(i)
                            x0b = plsc.load_gather(x0_t, [iv])
                            x1b = plsc.load_gather(x1_t, [iv])
                            dxav = plsc.load_gather(dxa_t, [iv])
                            dxbv = plsc.load_gather(dxb_t, [iv])

                            @plsc.parallel_loop(0, W, _L, unroll=4)
                            def do_q(j0):
                                yb0v = yb0_t[pl.ds(j0, _L)]
                                wdv = wd_t[pl.ds(j0, _L)]
                                dycv = dyc_t[pl.ds(j0, _L)]
                                yb1v = jnp.minimum(yb0v + W, (H - 1) * W)
                                wav = dycv - wdv
                                Ia = plsc.load_gather(img_v, [yb0v + x0b])
                                Ib = plsc.load_gather(img_v, [yb1v + x0b])
                                Ic = plsc.load_gather(img_v, [yb0v + x1b])
                                Id = plsc.load_gather(img_v, [yb1v + x1b])
                                u = wav * Ia + dycv * Ib
                                v = wav * Ic + wdv * Id
                                ob[pl.ds(r * W + j0, _L)] = (dxav * u
                                                             + dxbv * v)
                        pltpu.async_copy(ob, hbm, sem)
                    return carry2
                lax.fori_loop(0, NCH2, do_pair, 0)

            def slow_path():
                def do_pair(cc2, carry2):
                    for half, (ob, sem) in enumerate(((ob0, sem0),
                                                      (ob1, sem1))):
                        cc = cc2 * 2 + half
                        first = (k == 0) & (cc2 == 0)
                        hbm = store_chunk(ob, sem, cc, first)

                        def do_row(r, carry3):
                            i = cc * RC + r
                            xgv = plsc.load_gather(g_v, [splat(i)])
                            av = tv[0] * xgv + tv[2]
                            bv = tv[3] * xgv + tv[5]
                            cv = tv[6] * xgv + onev
                            for qq in range(NQ):
                                j0 = qq * _L
                                yv = g_v[pl.ds(j0, _L)]
                                xl = av + tv[1] * yv
                                yl = bv + tv[4] * yv
                                om = cv + tv[7] * yv
                                xs = xl / om
                                ys = yl / om
                                xm = ((xs + 1.0) * fx1) * 0.5
                                ym = ((ys + 1.0) * fy1) * 0.5
                                x0 = _floor_fix(xm)
                                y0 = _floor_fix(ym)
                                x0c = jnp.clip(x0, 0, W - 1)
                                x1c = jnp.clip(x0 + 1, 0, W - 1)
                                y0c = jnp.clip(y0, 0, H - 1)
                                y1c = jnp.clip(y0 + 1, 0, H - 1)
                                yb0v = y0c * W
                                yb1v = y1c * W
                                Ia = plsc.load_gather(img_v, [yb0v + x0c])
                                Ib = plsc.load_gather(img_v, [yb1v + x0c])
                                Ic = plsc.load_gather(img_v, [yb0v + x1c])
                                Id = plsc.load_gather(img_v, [yb1v + x1c])
                                x0f = x0c.astype(jnp.float32)
                                x1f = x1c.astype(jnp.float32)
                                y0f = y0c.astype(jnp.float32)
                                y1f = y1c.astype(jnp.float32)
                                wa = (x1f - xm) * (y1f - ym)
                                wb = (x1f - xm) * (y1f - y0f)
                                wc = (xm - x0f) * (y1f - ym)
                                wd = (xm - x0f) * (ym - y0f)
                                val = wa * Ia + wb * Ib + wc * Ic + wd * Id
                                ob[pl.ds(r * W + j0, _L)] = val
                            return carry3
                        lax.fori_loop(0, RC, do_row, carry2)
                        pltpu.async_copy(ob, hbm, sem)
                    return carry2
                lax.fori_loop(0, NCH2, do_pair, 0)

            pl.when(separable)(fast_path)
            pl.when(jnp.logical_not(separable))(slow_path)
            return carry
        lax.fori_loop(0, per, do_image, 0)
        last_q = wid * per + (per - 1)
        for ob, sem in ((ob0, sem0), (ob1, sem1)):
            pltpu.make_async_copy(
                ob, out.at[last_q, pl.ds(0, CB)], sem).wait()

    return body


def _sc_call(imgs, ths, grid, TM, C, H, W, nq):
    per = nq // _NW
    RC = 16
    kfn = pl.kernel(
        _make_body(TM, C, per, H, W, RC),
        out_type=jax.ShapeDtypeStruct((nq, H * W), jnp.float32),
        mesh=plsc.VectorSubcoreMesh(core_axis_name="c", subcore_axis_name="s"),
        compiler_params=pltpu.CompilerParams(needs_layout_passes=False),
        scratch_types=[
            pltpu.VMEM((H * W,), jnp.float32),
            pltpu.VMEM((TM * C, 8), jnp.float32),
            pltpu.VMEM((W,), jnp.float32),
            pltpu.VMEM((RC * W,), jnp.float32),
            pltpu.VMEM((RC * W,), jnp.float32),
            pltpu.VMEM((W,), jnp.int32),
            pltpu.VMEM((W,), jnp.int32),
            pltpu.VMEM((W,), jnp.float32),
            pltpu.VMEM((W,), jnp.float32),
            pltpu.VMEM((W,), jnp.int32),
            pltpu.VMEM((W,), jnp.float32),
            pltpu.VMEM((W,), jnp.float32),
            pltpu.SemaphoreType.DMA,
            pltpu.SemaphoreType.DMA,
        ],
    )
    return kfn(imgs, ths, grid)


def _make_tc_body(H, W, TM, C, q0):
    fx1 = float(W - 2)
    fy1 = float(H - 2)

    def body(img_ref, th_ref, g_ref, out_ref):
        g = g_ref[0, :]
        q = q0 + pl.program_id(0)
        ti = (q % TM) * C + (q // TM) % C
        t0 = th_ref[ti, 0]
        t2 = th_ref[ti, 2]
        t4 = th_ref[ti, 4]
        t5 = th_ref[ti, 5]
        xs = t0 * g + t2
        xm = ((xs + 1.0) * fx1) * 0.5
        x0 = jnp.floor(xm).astype(jnp.int32)
        x0c = jnp.clip(x0, 0, W - 1)
        x1c = jnp.clip(x0 + 1, 0, W - 1)
        dxa = x1c.astype(jnp.float32) - xm
        dxb = xm - x0c.astype(jnp.float32)
        ys = t4 * g + t5
        ym = ((ys + 1.0) * fy1) * 0.5
        y0 = jnp.floor(ym).astype(jnp.int32)
        y0c = jnp.clip(y0, 0, H - 1)
        y1c = jnp.clip(y0 + 1, 0, H - 1)
        y0f = y0c.astype(jnp.float32)
        y1f = y1c.astype(jnp.float32)
        wd = ym - y0f
        dyc = y1f - y0f
        wav = dyc - wd
        iota0 = lax.broadcasted_iota(jnp.int32, (W, W), 0)
        zero = jnp.zeros((W, W), jnp.float32)
        wxat = jnp.where(iota0 == x0c[None, :], dxa[None, :], zero)
        wxbt = jnp.where(iota0 == x1c[None, :], dxb[None, :], zero)
        wyat = (jnp.where(iota0 == y0c[None, :], wav[None, :], zero)
                + jnp.where(iota0 == y1c[None, :], dyc[None, :], zero))
        wybt = (jnp.where(iota0 == y0c[None, :], wav[None, :], zero)
                + jnp.where(iota0 == y1c[None, :], wd[None, :], zero))
        img = img_ref[0]
        pa = lax.dot_general(img, wxat, (((1,), (0,)), ((), ())),
                             precision=lax.Precision.HIGHEST,
                             preferred_element_type=jnp.float32)
        pb = lax.dot_general(img, wxbt, (((1,), (0,)), ((), ())),
                             precision=lax.Precision.HIGHEST,
                             preferred_element_type=jnp.float32)
        o = (lax.dot_general(pa, wyat, (((0,), (0,)), ((), ())),
                             precision=lax.Precision.HIGHEST,
                             preferred_element_type=jnp.float32)
             + lax.dot_general(pb, wybt, (((0,), (0,)), ((), ())),
                               precision=lax.Precision.HIGHEST,
                               preferred_element_type=jnp.float32))
        out_ref[0] = o

    return body


def _tc_call(imgs3, ths, grid2, TM, C, H, W, q0, nq):
    return pl.pallas_call(
        _make_tc_body(H, W, TM, C, q0),
        grid=(nq,),
        in_specs=[
            pl.BlockSpec((1, H, W), lambda i: ((q0 + i) // TM, 0, 0)),
            pl.BlockSpec((TM * C, 8), lambda i: (0, 0)),
            pl.BlockSpec((1, W), lambda i: (0, 0)),
        ],
        out_specs=pl.BlockSpec((1, H, W), lambda i: (i, 0, 0)),
        out_shape=jax.ShapeDtypeStruct((nq, H, W), jnp.float32),
    )(imgs3, ths, grid2)


def kernel(inputs, wt_pers):
    B, C, H, W = inputs.shape
    TM = wt_pers.shape[0]
    ntasks = B * C * TM
    imgs3 = inputs.reshape(B * C, H, W)
    imgs = inputs.reshape(B * C, H * W)
    ths = wt_pers.reshape(TM * C, 8)
    ths = ths.astype(jnp.bfloat16).astype(jnp.float32)
    grid = jnp.linspace(-1.0, 1.0, W, dtype=jnp.float32)
    grid = grid.astype(jnp.bfloat16).astype(jnp.float32)
    grid2 = grid.reshape(1, W)

    all_sep = jnp.all((ths[:, 1] == 0.0) & (ths[:, 3] == 0.0)
                      & (ths[:, 6] == 0.0) & (ths[:, 7] == 0.0))

    K = (ntasks // (3 * _NW)) * _NW
    if K == 0 or K == ntasks or (K % TM) != 0:
        K = ntasks

    def hybrid(_):
        out_sc = _sc_call(imgs, ths, grid, TM, C, H, W, K)
        out_tc = _tc_call(imgs3, ths, grid2, TM, C, H, W, K, ntasks - K)
        return jnp.concatenate(
            [out_sc, out_tc.reshape(ntasks - K, H * W)], axis=0)

    def sc_only(_):
        return _sc_call(imgs, ths, grid, TM, C, H, W, ntasks)

    if K == ntasks:
        out = sc_only(0)
    else:
        out = lax.cond(all_sep, hybrid, sc_only, 0)
    return out.reshape(B, C * TM, H, W)
